# baseline (speedup 1.0000x reference)
import jax
import jax.numpy as jnp
from jax import lax
from jax.experimental import pallas as pl
from jax.experimental.pallas import tpu as pltpu
from jax.experimental.pallas import tpu_sc as plsc

_C = 384
_HW = 3136
_NB = 3

_NC = 2
_NS = 16
_NW = _NC * _NS
_RPW = 80
_CH = 16
_NCHUNK = _RPW // _CH
_SC_BASE = 12288 - _RPW * _NW


def _tc_body(x_hbm, w_ref, out_ref, buf, sem):
    b = pl.program_id(0)
    nb = pl.num_programs(0)

    @pl.when(b == 0)
    def _():
        for j in range(_NB - 1):
            pltpu.make_async_copy(x_hbm.at[j], buf.at[j], sem.at[j]).start()

    pre = b + _NB - 1
    slot_pre = jax.lax.rem(pre, _NB)

    @pl.when(pre < nb)
    def _():
        for j in range(_NB):

            @pl.when(slot_pre == j)
            def _():
                pltpu.make_async_copy(x_hbm.at[pre], buf.at[j], sem.at[j]).start()

    slot = jax.lax.rem(b, _NB)
    for j in range(_NB):

        @pl.when(slot == j)
        def _():
            pltpu.make_async_copy(x_hbm.at[b], buf.at[j], sem.at[j]).wait()

    out_ref[0] = buf[slot, :3] * 2.0


def _sc_probe(x_hbm, out_hbm, buf0, buf1, sem0, sem1):
    wid = lax.axis_index("s") * _NC + lax.axis_index("c")
    base = _SC_BASE + wid * _RPW
    bufs = (buf0, buf1)
    sems = (sem0, sem1)

    pltpu.make_async_copy(x_hbm.at[pl.ds(base, _CH)], buf0, sem0).start()

    def step(i, carry):
        del carry
        slot = lax.rem(i, 2)
        nxt = lax.rem(i + 1, 2)

        @pl.when(i + 1 < _NCHUNK)
        def _():
            for j in range(2):

                @pl.when(nxt == j)
                def _():
                    pltpu.make_async_copy(
                        x_hbm.at[pl.ds(base + (i + 1) * _CH, _CH)],
                        bufs[j], sems[j],
                    ).start()

        for j in range(2):

            @pl.when(slot == j)
            def _():
                pltpu.make_async_copy(
                    x_hbm.at[pl.ds(base + i * _CH, _CH)],
                    bufs[j], sems[j],
                ).wait()

        return 0

    lax.fori_loop(0, _NCHUNK, step, 0)

    @pl.when(wid < 12)
    def _():
        pltpu.sync_copy(buf0.at[pl.ds(0, 8)], out_hbm.at[pl.ds(wid * 8, 8)])


@jax.jit
def kernel(x, w):
    b, c, h, wd = x.shape
    x3 = x.reshape(b, c, h * wd)
    tc_out = pl.pallas_call(
        _tc_body,
        grid=(26,),
        in_specs=[
            pl.BlockSpec(memory_space=pl.ANY),
            pl.BlockSpec(memory_space=pltpu.SMEM),
        ],
        out_specs=pl.BlockSpec((1, 3, h * wd), lambda i: (i, 0, 0)),
        out_shape=jax.ShapeDtypeStruct((26, 3, h * wd), x.dtype),
        scratch_shapes=[
            pltpu.VMEM((_NB, c, h * wd), jnp.float32),
            pltpu.SemaphoreType.DMA((_NB,)),
        ],
    )(x3, w)
    mesh = plsc.VectorSubcoreMesh(core_axis_name="c", subcore_axis_name="s")
    sc_out = pl.kernel(
        _sc_probe,
        mesh=mesh,
        out_type=jax.ShapeDtypeStruct((96, h * wd), jnp.float32),
        scratch_types=[
            pltpu.VMEM((_CH, _HW), jnp.float32),
            pltpu.VMEM((_CH, _HW), jnp.float32),
            pltpu.SemaphoreType.DMA,
            pltpu.SemaphoreType.DMA,
        ],
    )(x.reshape(b * c, h * wd))
    tail = sc_out.reshape(32, 3, h * wd)[:6]
    out = jnp.concatenate([tc_out, tail], axis=0)
    return out.reshape(b, 3, h, wd)


# speedup vs baseline: 3.2015x; 3.2015x over previous
import jax
import jax.numpy as jnp
from jax.experimental import pallas as pl
from jax.experimental.pallas import tpu as pltpu

_C = 384
_HW = 3136
_NB = 3


def _body(x_hbm, w_ref, out_ref, buf, sem):
    b = pl.program_id(0)
    nb = pl.num_programs(0)

    @pl.when(b == 0)
    def _():
        for j in range(_NB - 1):
            pltpu.make_async_copy(x_hbm.at[j], buf.at[j], sem.at[j]).start()

    pre = b + _NB - 1
    slot_pre = jax.lax.rem(pre, _NB)

    @pl.when(pre < nb)
    def _():
        for j in range(_NB):

            @pl.when(slot_pre == j)
            def _():
                pltpu.make_async_copy(x_hbm.at[pre], buf.at[j], sem.at[j]).start()

    slot = jax.lax.rem(b, _NB)
    for j in range(_NB):

        @pl.when(slot == j)
        def _():
            pltpu.make_async_copy(x_hbm.at[b], buf.at[j], sem.at[j]).wait()

    xv = buf[slot]
    y = jnp.sum(xv, axis=1)
    yr = y.reshape(1, _C)
    iota = jax.lax.broadcasted_iota(jnp.int32, (1, _C), 1)
    scale = 1.0 / _HW
    w0 = w_ref[0] * scale
    w1 = w_ref[1] * scale
    w2 = w_ref[2] * scale
    yprev = jnp.where(iota == 0, 0.0, pltpu.roll(yr, 1, axis=1))
    ynext = jnp.where(iota == _C - 1, 0.0, pltpu.roll(yr, _C - 1, axis=1))
    s = w0 * yprev + w1 * yr + w2 * ynext
    cur = s
    for k in range(3):
        m = jnp.max(cur)
        idx_k = jnp.min(jnp.where(cur == m, iota, _C))
        out_ref[0, pl.ds(k, 1)] = buf[slot, pl.ds(idx_k, 1)]
        cur = jnp.where(iota == idx_k, -jnp.inf, cur)


@jax.jit
def kernel(x, w):
    b, c, h, wd = x.shape
    x3 = x.reshape(b, c, h * wd)
    out = pl.pallas_call(
        _body,
        grid=(b,),
        in_specs=[
            pl.BlockSpec(memory_space=pl.ANY),
            pl.BlockSpec(memory_space=pltpu.SMEM),
        ],
        out_specs=pl.BlockSpec((1, 3, h * wd), lambda i: (i, 0, 0)),
        out_shape=jax.ShapeDtypeStruct((b, 3, h * wd), x.dtype),
        scratch_shapes=[
            pltpu.VMEM((_NB, c, h * wd), jnp.float32),
            pltpu.SemaphoreType.DMA((_NB,)),
        ],
    )(x3, w)
    return out.reshape(b, 3, h, wd)
